# baseline (speedup 1.0000x reference)
import functools

import jax
import jax.numpy as jnp
from jax import lax
from jax.experimental import pallas as pl
from jax.experimental import pallas as _pl
from jax.experimental.pallas import tpu as pltpu
from jax.experimental.pallas import tpu_sc as plsc

_NC = 2
_NS = 16
_NW = _NC * _NS


def _sc_gather_rows(baseline_log, idx_b, genotypes, idx_g):
    n_c, n_genes = baseline_log.shape
    n_d, n_var = genotypes.shape
    n_v = idx_b.shape[0]
    n_tasks = n_c + n_d
    n_rounds = (n_tasks + _NW - 1) // _NW
    mesh = plsc.VectorSubcoreMesh(core_axis_name="c", subcore_axis_name="s")

    @functools.partial(
        pl.kernel,
        mesh=mesh,
        out_type=[
            jax.ShapeDtypeStruct((n_c, n_v), jnp.float32),
            jax.ShapeDtypeStruct((n_d, n_v), jnp.float32),
        ],
        scratch_types=[
            pltpu.VMEM((n_genes,), jnp.float32),
            pltpu.VMEM((n_v,), jnp.int32),
            pltpu.VMEM((n_v,), jnp.int32),
            pltpu.VMEM((n_v,), jnp.float32),
        ],
        compiler_params=pltpu.CompilerParams(use_tc_tiling_on_sc=False,
                                             needs_layout_passes=False),
    )
    def gather_kernel(bl, ib, gen, ig, ob, og, rowv, ibv, igv, outv):
        wid = lax.axis_index("s") * _NC + lax.axis_index("c")
        pltpu.sync_copy(ib, ibv)
        pltpu.sync_copy(ig, igv)

        @pl.loop(0, n_rounds)
        def _round(r):
            t = wid + r * _NW

            @pl.when(t < n_c)
            def _():
                pltpu.sync_copy(bl.at[t], rowv)

                @plsc.parallel_loop(0, n_v, step=16, unroll=8)
                def _(i):
                    outv[pl.ds(i, 16)] = plsc.load_gather(rowv, [ibv[pl.ds(i, 16)]])

                pltpu.sync_copy(outv, ob.at[t])

            @pl.when((t >= n_c) & (t < n_tasks))
            def _():
                td = t - n_c
                pltpu.sync_copy(gen.at[td], rowv.at[pl.ds(0, n_var)])

                @plsc.parallel_loop(0, n_v, step=16, unroll=8)
                def _(i):
                    outv[pl.ds(i, 16)] = plsc.load_gather(rowv, [igv[pl.ds(i, 16)]])

                pltpu.sync_copy(outv, og.at[td])

    return gather_kernel(baseline_log, idx_b, genotypes, idx_g)


def _tc_body(b_ref, g_ref, fc_ref, lib_ref, o_ref):
    b = b_ref[...]
    g = g_ref[...]
    fc = fc_ref[...]
    libt = lib_ref[...].T
    x = b[:, None, :] + g[None, :, :] * fc[:, None, :]
    o_ref[...] = jnp.exp(x) * libt[:, :, None]


def kernel(fc_log, genotypes, expression_obs, variantxgene_to_gene,
           local_variant_to_local_variantxgene_selector, variantxgene_to_local_gene,
           lib, baseline_log, dispersion_log):
    n_clusters, n_vxg = fc_log.shape
    n_donors = genotypes.shape[0]

    b, g = _sc_gather_rows(baseline_log, variantxgene_to_gene,
                           genotypes, local_variant_to_local_variantxgene_selector)

    vb = 2048
    out = pl.pallas_call(
        _tc_body,
        grid=(n_vxg // vb,),
        in_specs=[
            pl.BlockSpec((n_clusters, vb), lambda i: (0, i)),
            pl.BlockSpec((n_donors, vb), lambda i: (0, i)),
            pl.BlockSpec((n_clusters, vb), lambda i: (0, i)),
            pl.BlockSpec((n_donors, n_clusters), lambda i: (0, 0)),
        ],
        out_specs=pl.BlockSpec((n_clusters, n_donors, vb), lambda i: (0, 0, i)),
        out_shape=jax.ShapeDtypeStruct((n_clusters, n_donors, n_vxg), jnp.float32),
    )(b, g, fc_log, lib)
    return jnp.transpose(out, (1, 0, 2))


# speedup vs baseline: 1.0055x; 1.0055x over previous
import functools

import jax
import jax.numpy as jnp
from jax import lax
from jax.experimental import pallas as pl
from jax.experimental import pallas as _pl
from jax.experimental.pallas import tpu as pltpu
from jax.experimental.pallas import tpu_sc as plsc

_NC = 2
_NS = 16
_NW = _NC * _NS


def _sc_gather_rows(baseline_log, idx_b, genotypes, idx_g):
    n_c, n_genes = baseline_log.shape
    n_d, n_var = genotypes.shape
    n_v = idx_b.shape[0]
    n_tasks = n_c + n_d
    n_rounds = (n_tasks + _NW - 1) // _NW
    mesh = plsc.VectorSubcoreMesh(core_axis_name="c", subcore_axis_name="s")

    @functools.partial(
        pl.kernel,
        mesh=mesh,
        out_type=[
            jax.ShapeDtypeStruct((n_c, n_v), jnp.float32),
            jax.ShapeDtypeStruct((n_d, n_v), jnp.float32),
        ],
        scratch_types=[
            pltpu.VMEM((n_genes,), jnp.float32),
            pltpu.VMEM((n_v,), jnp.int32),
            pltpu.VMEM((n_v,), jnp.int32),
            pltpu.VMEM((n_v,), jnp.float32),
        ],
        compiler_params=pltpu.CompilerParams(use_tc_tiling_on_sc=False,
                                             needs_layout_passes=False),
    )
    def gather_kernel(bl, ib, gen, ig, ob, og, rowv, ibv, igv, outv):
        wid = lax.axis_index("s") * _NC + lax.axis_index("c")
        pltpu.sync_copy(ib, ibv)
        pltpu.sync_copy(ig, igv)

        @pl.loop(0, n_rounds)
        def _round(r):
            t = wid + r * _NW

            @pl.when(t < n_c)
            def _():
                pltpu.sync_copy(bl.at[t], rowv)

                @plsc.parallel_loop(0, n_v, step=16, unroll=8)
                def _(i):
                    outv[pl.ds(i, 16)] = plsc.load_gather(rowv, [ibv[pl.ds(i, 16)]])

                pltpu.sync_copy(outv, ob.at[t])

            @pl.when((t >= n_c) & (t < n_tasks))
            def _():
                td = t - n_c
                pltpu.sync_copy(gen.at[td], rowv.at[pl.ds(0, n_var)])

                @plsc.parallel_loop(0, n_v, step=16, unroll=8)
                def _(i):
                    outv[pl.ds(i, 16)] = plsc.load_gather(rowv, [igv[pl.ds(i, 16)]])

                pltpu.sync_copy(outv, og.at[td])

    return gather_kernel(baseline_log, idx_b, genotypes, idx_g)


def _tc_body(b_ref, g_ref, fc_ref, lib_ref, o_ref):
    b = b_ref[...]
    g = g_ref[...]
    fc = fc_ref[...]
    libt = lib_ref[...].T
    x = b[:, None, :] + g[None, :, :] * fc[:, None, :]
    o_ref[...] = jnp.exp(x) * libt[:, :, None]


def kernel(fc_log, genotypes, expression_obs, variantxgene_to_gene,
           local_variant_to_local_variantxgene_selector, variantxgene_to_local_gene,
           lib, baseline_log, dispersion_log):
    n_clusters, n_vxg = fc_log.shape
    n_donors = genotypes.shape[0]

    b, g = _sc_gather_rows(baseline_log, variantxgene_to_gene,
                           genotypes, local_variant_to_local_variantxgene_selector)

    vb = 1024
    out = pl.pallas_call(
        _tc_body,
        grid=(n_vxg // vb,),
        in_specs=[
            pl.BlockSpec((n_clusters, vb), lambda i: (0, i)),
            pl.BlockSpec((n_donors, vb), lambda i: (0, i)),
            pl.BlockSpec((n_clusters, vb), lambda i: (0, i)),
            pl.BlockSpec((n_donors, n_clusters), lambda i: (0, 0)),
        ],
        out_specs=pl.BlockSpec((n_clusters, n_donors, vb), lambda i: (0, 0, i)),
        out_shape=jax.ShapeDtypeStruct((n_clusters, n_donors, n_vxg), jnp.float32),
    )(b, g, fc_log, lib)
    return jnp.transpose(out, (1, 0, 2))
